# baseline (speedup 1.0000x reference)
import functools

import jax
import jax.numpy as jnp
from jax import lax
from jax.experimental import pallas as pl
from jax.experimental.pallas import tpu as pltpu
from jax.experimental.pallas import tpu_sc as plsc

N = 10000
E = 320000
D = 128
H = 128

NC = 2
NS = 16
NW = NC * NS
EC = E // NW
CH = 40
NCHUNK = EC // CH
NCPB = 10
NSB = NCHUNK // NCPB
IDXW = NCPB * CH
NPAD = 10112
STRIPE = NPAD // NS
RB = 1000
DEN_B = 10
FRB = 128
NFB = NPAD // FRB


def _proj_body(x_ref, ws_ref, wd_ref, avs_ref, avd_ref,
               xs_ref, asrc_ref, adst_ref, mb_ref):
    i = pl.program_id(0)
    xb = x_ref[...]
    xs = jnp.dot(xb, ws_ref[...], preferred_element_type=jnp.float32)
    xd = jnp.dot(xb, wd_ref[...], preferred_element_type=jnp.float32)
    xs_ref[...] = xs
    a_s = jnp.sum(xs * avs_ref[...], axis=1, keepdims=True)
    a_d = jnp.sum(xd * avd_ref[...], axis=1, keepdims=True)
    asrc_ref[...] = a_s
    adst_ref[...] = a_d
    bm = jnp.concatenate(
        [jnp.full((1, H), jnp.max(a_s), jnp.float32),
         jnp.full((1, H), jnp.max(a_d), jnp.float32)], axis=0)

    @pl.when(i == 0)
    def _():
        mb_ref[...] = bm

    @pl.when(i != 0)
    def _():
        mb_ref[...] = jnp.maximum(mb_ref[...], bm)


_proj_call = pl.pallas_call(
    _proj_body,
    grid=(N // RB,),
    in_specs=[
        pl.BlockSpec((RB, D), lambda i: (i, 0)),
        pl.BlockSpec((D, H), lambda i: (0, 0)),
        pl.BlockSpec((D, H), lambda i: (0, 0)),
        pl.BlockSpec((1, H), lambda i: (0, 0)),
        pl.BlockSpec((1, H), lambda i: (0, 0)),
    ],
    out_specs=[
        pl.BlockSpec((RB, H), lambda i: (i, 0)),
        pl.BlockSpec((RB, 1), lambda i: (i, 0)),
        pl.BlockSpec((RB, 1), lambda i: (i, 0)),
        pl.BlockSpec((2, H), lambda i: (0, 0)),
    ],
    out_shape=[
        jax.ShapeDtypeStruct((N, H), jnp.float32),
        jax.ShapeDtypeStruct((N, 1), jnp.float32),
        jax.ShapeDtypeStruct((N, 1), jnp.float32),
        jax.ShapeDtypeStruct((2, H), jnp.float32),
    ],
)


def _sc_body(asrc_hbm, adst_hbm, src_hbm, dst_hbm, xs_hbm, mb_hbm,
             num_out, den_out,
             asrc_v, adst_v, cur_src, cur_dst, nxt_src, nxt_dst,
             srcb0, srcb1, dstb0, dstb1, exb,
             den_v, rows0, rows1, mb_v,
             num_sh, sem_g0, sem_g1, sem_s0, sem_s1, sem_i):
    c = lax.axis_index("c")
    s = lax.axis_index("s")
    w = c * NS + s
    rows = (rows0, rows1)
    srcb = (srcb0, srcb1)
    dstb = (dstb0, dstb1)
    sem_g = (sem_g0, sem_g1)
    sem_s = (sem_s0, sem_s1)

    pltpu.sync_copy(asrc_hbm, asrc_v)
    pltpu.sync_copy(adst_hbm, adst_v)
    pltpu.sync_copy(mb_hbm, mb_v)

    zeros16 = jnp.zeros((16,), jnp.float32)

    @pl.loop(0, DEN_B)
    def _zero_den(i):
        @pl.loop(0, 1024 // 16)
        def _(g):
            den_v[i, pl.ds(g * 16, 16)] = zeros16

    @pl.loop(0, CH)
    def _zero_rows(r):
        for h in range(H // 16):
            rows0[r, pl.ds(h * 16, 16)] = zeros16

    base = s * STRIPE
    nfull, rem = STRIPE // CH, STRIPE % CH

    @pl.loop(0, nfull)
    def _zero_sh(i):
        pltpu.sync_copy(rows0, num_sh.at[pl.ds(base + i * CH, CH)])

    if rem:
        pltpu.sync_copy(rows0.at[pl.ds(0, rem)],
                        num_sh.at[pl.ds(base + nfull * CH, rem)])

    plsc.subcore_barrier()

    m0 = mb_v[pl.ds(0, 16)] + mb_v[pl.ds(H, 16)]
    mb = jnp.maximum(m0, 0.2 * m0)

    iota16 = lax.iota(jnp.int32, 16)

    def start_idx(sbn):
        pltpu.async_copy(src_hbm.at[w, sbn], nxt_src, sem_i)
        pltpu.async_copy(dst_hbm.at[w, sbn], nxt_dst, sem_i)

    def wait_idx():
        pltpu.make_async_copy(src_hbm.at[w, 0], nxt_src, sem_i).wait()
        pltpu.make_async_copy(dst_hbm.at[w, 0], nxt_dst, sem_i).wait()

    def adopt_idx():
        for g in range(IDXW // 16):
            cur_src[pl.ds(g * 16, 16)] = nxt_src[pl.ds(g * 16, 16)]
            cur_dst[pl.ds(g * 16, 16)] = nxt_dst[pl.ds(g * 16, 16)]

    def stage(b, q):
        for o in (0, 16, 24):
            srcb[b][pl.ds(o, 16)] = cur_src[pl.ds(q * CH + o, 16)]
            dstb[b][pl.ds(o, 16)] = cur_dst[pl.ds(q * CH + o, 16)]

    def start_gather(b):
        pltpu.async_copy(xs_hbm.at[srcb[b]], rows[b], sem_g[b])

    def wait_gather(b):
        pltpu.make_async_copy(xs_hbm.at[srcb[b]], rows[b], sem_g[b]).wait()

    def start_scatter(b):
        pltpu.async_copy(rows[b], num_sh.at[dstb[b]], sem_s[b], add=True)

    def wait_scatter(b):
        pltpu.make_async_copy(rows[b], num_sh.at[dstb[b]], sem_s[b]).wait()

    tail_mask = iota16 >= 8

    def score(b):
        offs = (0, 16, 24)
        masks = (None, None, tail_mask)
        for o, m in zip(offs, masks):
            sidx = srcb[b][pl.ds(o, 16)]
            didx = dstb[b][pl.ds(o, 16)]
            a1 = plsc.load_gather(asrc_v, [sidx])
            a2 = plsc.load_gather(adst_v, [didx])
            t = a1 + a2
            e = jnp.maximum(t, 0.2 * t)
            ex = jnp.exp(e - mb)
            exb[pl.ds(o, 16)] = ex
            dblk = lax.shift_right_logical(didx, 10)
            dpos = lax.bitwise_and(didx, 1023)
            if m is None:
                plsc.addupdate_scatter(den_v, [dblk, dpos], ex)
            else:
                plsc.addupdate_scatter(den_v, [dblk, dpos], ex, mask=m)

    def scale(b):
        rows_b = rows[b]
        for r in range(CH):
            rv = jnp.full((16,), r, jnp.int32)
            sc = plsc.load_gather(exb, [rv])
            for h in range(H // 16):
                rows_b[r, pl.ds(h * 16, 16)] = (
                    rows_b[r, pl.ds(h * 16, 16)] * sc)

    start_idx(0)
    wait_idx()
    adopt_idx()
    start_idx(1)
    stage(0, 0)
    start_gather(0)

    @pl.loop(0, NCHUNK - 2, step=2)
    def _pipe(jj):
        for b in (0, 1):
            j = jj + b
            nb = 1 - b
            j1 = j + 1
            sb1 = lax.div(j1, NCPB)
            q1 = j1 - sb1 * NCPB
            score(b)
            if b == 0:
                @pl.when(jj > 0)
                def _():
                    wait_scatter(nb)
            else:
                wait_scatter(nb)
            wait_gather(b)

            @pl.when(q1 == 0)
            def _():
                wait_idx()
                adopt_idx()

                @pl.when(sb1 + 1 < NSB)
                def _():
                    start_idx(sb1 + 1)

            stage(nb, q1)
            start_gather(nb)
            scale(b)
            start_scatter(b)

    score(0)
    wait_scatter(1)
    wait_gather(0)
    stage(1, NCPB - 1)
    start_gather(1)
    scale(0)
    start_scatter(0)
    score(1)
    wait_scatter(0)
    wait_gather(1)
    scale(1)
    start_scatter(1)
    wait_scatter(1)

    for i in range(DEN_B):
        pltpu.async_copy(den_v.at[i], den_out.at[i, w], sem_i)
    for i in range(DEN_B):
        pltpu.make_async_copy(den_v.at[i], den_out.at[i, w], sem_i).wait()
    plsc.subcore_barrier()
    pltpu.sync_copy(num_sh.at[pl.ds(s * STRIPE, STRIPE)],
                    num_out.at[c, pl.ds(s * STRIPE, STRIPE)])


_sc_call = functools.partial(
    pl.kernel,
    out_type=(
        jax.ShapeDtypeStruct((NC, NPAD, H), jnp.float32),
        jax.ShapeDtypeStruct((DEN_B, NW, 1024), jnp.float32),
    ),
    mesh=plsc.VectorSubcoreMesh(core_axis_name="c", subcore_axis_name="s",
                                num_cores=NC, num_subcores=NS),
    compiler_params=pltpu.CompilerParams(needs_layout_passes=False),
    scratch_types=[
        pltpu.VMEM((N,), jnp.float32),
        pltpu.VMEM((N,), jnp.float32),
        pltpu.VMEM((IDXW,), jnp.int32),
        pltpu.VMEM((IDXW,), jnp.int32),
        pltpu.VMEM((IDXW,), jnp.int32),
        pltpu.VMEM((IDXW,), jnp.int32),
        pltpu.VMEM((CH,), jnp.int32),
        pltpu.VMEM((CH,), jnp.int32),
        pltpu.VMEM((CH,), jnp.int32),
        pltpu.VMEM((CH,), jnp.int32),
        pltpu.VMEM((CH,), jnp.float32),
        pltpu.VMEM((DEN_B, 1024), jnp.float32),
        pltpu.VMEM((CH, H), jnp.float32),
        pltpu.VMEM((CH, H), jnp.float32),
        pltpu.VMEM((2 * H,), jnp.float32),
        pltpu.VMEM_SHARED((NPAD, H), jnp.float32),
        pltpu.SemaphoreType.DMA,
        pltpu.SemaphoreType.DMA,
        pltpu.SemaphoreType.DMA,
        pltpu.SemaphoreType.DMA,
        pltpu.SemaphoreType.DMA,
    ],
)(_sc_body)


def _fin_body(num_ref, den_ref, bias_ref, out_ref):
    n = num_ref[0] + num_ref[1]
    dsum = jnp.sum(den_ref[0], axis=0)
    out_ref[...] = jnp.maximum(
        n / (dsum[:, None] + 1e-16) + bias_ref[...], 0.0)


_fin_call = pl.pallas_call(
    _fin_body,
    grid=(NFB,),
    in_specs=[
        pl.BlockSpec((NC, FRB, H), lambda i: (0, i, 0)),
        pl.BlockSpec((1, NW, FRB), lambda i: (i // 8, 0, i % 8)),
        pl.BlockSpec((1, H), lambda i: (0, 0)),
    ],
    out_specs=pl.BlockSpec((FRB, H), lambda i: (i, 0)),
    out_shape=jax.ShapeDtypeStruct((N, H), jnp.float32),
)


def kernel(x, edge_index, W_src, W_dst, att_src, att_dst, bias):
    xs, a_src, a_dst, mb2 = _proj_call(
        x, W_src, W_dst, att_src.reshape(1, H), att_dst.reshape(1, H))
    src = edge_index[0].reshape(NW, NSB, IDXW)
    dst = edge_index[1].reshape(NW, NSB, IDXW)
    num, den = _sc_call(a_src.reshape(N), a_dst.reshape(N), src, dst, xs,
                        mb2.reshape(2 * H))
    return _fin_call(num, den, bias.reshape(1, H))


# speedup vs baseline: 1.0058x; 1.0058x over previous
import functools

import jax
import jax.numpy as jnp
from jax import lax
from jax.experimental import pallas as pl
from jax.experimental.pallas import tpu as pltpu
from jax.experimental.pallas import tpu_sc as plsc

N = 10000
E = 320000
D = 128
H = 128

NC = 2
NS = 16
NW = NC * NS
EC = E // NW
CH = 40
NCHUNK = EC // CH
NCPB = 10
NSB = NCHUNK // NCPB
IDXW = NCPB * CH
NPAD = 10112
STRIPE = NPAD // NS
RB = 1000
DEN_B = 10
FRB = 128
NFB = NPAD // FRB


def _proj_body(x_ref, ws_ref, wd_ref, avs_ref, avd_ref,
               xs_ref, asrc_ref, adst_ref, mb_ref):
    i = pl.program_id(0)
    xb = x_ref[...]
    xs = jnp.dot(xb, ws_ref[...], preferred_element_type=jnp.float32)
    xd = jnp.dot(xb, wd_ref[...], preferred_element_type=jnp.float32)
    xs_ref[...] = xs
    a_s = jnp.sum(xs * avs_ref[...], axis=1, keepdims=True)
    a_d = jnp.sum(xd * avd_ref[...], axis=1, keepdims=True)
    asrc_ref[...] = a_s
    adst_ref[...] = a_d
    bm = jnp.concatenate(
        [jnp.full((1, H), jnp.max(a_s), jnp.float32),
         jnp.full((1, H), jnp.max(a_d), jnp.float32)], axis=0)

    @pl.when(i == 0)
    def _():
        mb_ref[...] = bm

    @pl.when(i != 0)
    def _():
        mb_ref[...] = jnp.maximum(mb_ref[...], bm)


_proj_call = pl.pallas_call(
    _proj_body,
    grid=(N // RB,),
    in_specs=[
        pl.BlockSpec((RB, D), lambda i: (i, 0)),
        pl.BlockSpec((D, H), lambda i: (0, 0)),
        pl.BlockSpec((D, H), lambda i: (0, 0)),
        pl.BlockSpec((1, H), lambda i: (0, 0)),
        pl.BlockSpec((1, H), lambda i: (0, 0)),
    ],
    out_specs=[
        pl.BlockSpec((RB, H), lambda i: (i, 0)),
        pl.BlockSpec((RB, 1), lambda i: (i, 0)),
        pl.BlockSpec((RB, 1), lambda i: (i, 0)),
        pl.BlockSpec((2, H), lambda i: (0, 0)),
    ],
    out_shape=[
        jax.ShapeDtypeStruct((N, H), jnp.float32),
        jax.ShapeDtypeStruct((N, 1), jnp.float32),
        jax.ShapeDtypeStruct((N, 1), jnp.float32),
        jax.ShapeDtypeStruct((2, H), jnp.float32),
    ],
)


def _sc_body(asrc_hbm, adst_hbm, src_hbm, dst_hbm, xs_hbm, mb_hbm,
             num_out, den_out,
             asrc_v, adst_v, cur_src, cur_dst, nxt_src, nxt_dst,
             srcb0, srcb1, dstb0, dstb1, exb,
             den_v, rows0, rows1, mb_v,
             num_sh, sem_g0, sem_g1, sem_s0, sem_s1, sem_i):
    c = lax.axis_index("c")
    s = lax.axis_index("s")
    w = c * NS + s
    rows = (rows0, rows1)
    srcb = (srcb0, srcb1)
    dstb = (dstb0, dstb1)
    sem_g = (sem_g0, sem_g1)
    sem_s = (sem_s0, sem_s1)

    pltpu.sync_copy(asrc_hbm, asrc_v)
    pltpu.sync_copy(adst_hbm, adst_v)
    pltpu.sync_copy(mb_hbm, mb_v)

    zeros16 = jnp.zeros((16,), jnp.float32)

    @pl.loop(0, DEN_B)
    def _zero_den(i):
        @pl.loop(0, 1024 // 16)
        def _(g):
            den_v[i, pl.ds(g * 16, 16)] = zeros16

    @pl.loop(0, CH)
    def _zero_rows(r):
        for h in range(H // 16):
            rows0[r, pl.ds(h * 16, 16)] = zeros16

    base = s * STRIPE
    nfull, rem = STRIPE // CH, STRIPE % CH

    @pl.loop(0, nfull)
    def _zero_sh(i):
        pltpu.sync_copy(rows0, num_sh.at[pl.ds(base + i * CH, CH)])

    if rem:
        pltpu.sync_copy(rows0.at[pl.ds(0, rem)],
                        num_sh.at[pl.ds(base + nfull * CH, rem)])

    plsc.subcore_barrier()

    m0 = mb_v[pl.ds(0, 16)] + mb_v[pl.ds(H, 16)]
    mb = jnp.maximum(m0, 0.2 * m0)

    iota16 = lax.iota(jnp.int32, 16)

    def start_idx(sbn):
        pltpu.async_copy(src_hbm.at[w, sbn], nxt_src, sem_i)
        pltpu.async_copy(dst_hbm.at[w, sbn], nxt_dst, sem_i)

    def wait_idx():
        pltpu.make_async_copy(src_hbm.at[w, 0], nxt_src, sem_i).wait()
        pltpu.make_async_copy(dst_hbm.at[w, 0], nxt_dst, sem_i).wait()

    def adopt_idx():
        for g in range(IDXW // 16):
            cur_src[pl.ds(g * 16, 16)] = nxt_src[pl.ds(g * 16, 16)]
            cur_dst[pl.ds(g * 16, 16)] = nxt_dst[pl.ds(g * 16, 16)]

    def stage(b, q):
        for o in (0, 16, 24):
            idxv = jnp.full((16,), q * CH + o, jnp.int32) + iota16
            srcb[b][pl.ds(o, 16)] = plsc.load_gather(cur_src, [idxv])
            dstb[b][pl.ds(o, 16)] = plsc.load_gather(cur_dst, [idxv])

    def start_gather(b):
        pltpu.async_copy(xs_hbm.at[srcb[b]], rows[b], sem_g[b])

    def wait_gather(b):
        pltpu.make_async_copy(xs_hbm.at[srcb[b]], rows[b], sem_g[b]).wait()

    def start_scatter(b):
        pltpu.async_copy(rows[b], num_sh.at[dstb[b]], sem_s[b], add=True)

    def wait_scatter(b):
        pltpu.make_async_copy(rows[b], num_sh.at[dstb[b]], sem_s[b]).wait()

    tail_mask = iota16 >= 8

    def score(b):
        offs = (0, 16, 24)
        masks = (None, None, tail_mask)
        for o, m in zip(offs, masks):
            sidx = srcb[b][pl.ds(o, 16)]
            didx = dstb[b][pl.ds(o, 16)]
            a1 = plsc.load_gather(asrc_v, [sidx])
            a2 = plsc.load_gather(adst_v, [didx])
            t = a1 + a2
            e = jnp.maximum(t, 0.2 * t)
            ex = jnp.exp(e - mb)
            exb[pl.ds(o, 16)] = ex
            dblk = lax.shift_right_logical(didx, 10)
            dpos = lax.bitwise_and(didx, 1023)
            if m is None:
                plsc.addupdate_scatter(den_v, [dblk, dpos], ex)
            else:
                plsc.addupdate_scatter(den_v, [dblk, dpos], ex, mask=m)

    def scale(b):
        rows_b = rows[b]

        @pl.loop(0, CH, unroll=4)
        def _scale(r):
            rv = jnp.full((16,), r, jnp.int32)
            sc = plsc.load_gather(exb, [rv])
            for h in range(H // 16):
                rows_b[r, pl.ds(h * 16, 16)] = (
                    rows_b[r, pl.ds(h * 16, 16)] * sc)

    start_idx(0)
    wait_idx()
    adopt_idx()
    start_idx(1)
    stage(0, 0)
    start_gather(0)

    @pl.loop(0, NCHUNK - 2, step=2)
    def _pipe(jj):
        for b in (0, 1):
            j = jj + b
            nb = 1 - b
            j1 = j + 1
            sb1 = lax.div(j1, NCPB)
            q1 = j1 - sb1 * NCPB
            score(b)
            if b == 0:
                @pl.when(jj > 0)
                def _():
                    wait_scatter(nb)
            else:
                wait_scatter(nb)
            wait_gather(b)

            @pl.when(q1 == 0)
            def _():
                wait_idx()
                adopt_idx()

                @pl.when(sb1 + 1 < NSB)
                def _():
                    start_idx(sb1 + 1)

            stage(nb, q1)
            start_gather(nb)
            scale(b)
            start_scatter(b)

    score(0)
    wait_scatter(1)
    wait_gather(0)
    stage(1, NCPB - 1)
    start_gather(1)
    scale(0)
    start_scatter(0)
    score(1)
    wait_scatter(0)
    wait_gather(1)
    scale(1)
    start_scatter(1)
    wait_scatter(1)

    for i in range(DEN_B):
        pltpu.async_copy(den_v.at[i], den_out.at[i, w], sem_i)
    for i in range(DEN_B):
        pltpu.make_async_copy(den_v.at[i], den_out.at[i, w], sem_i).wait()
    plsc.subcore_barrier()
    pltpu.sync_copy(num_sh.at[pl.ds(s * STRIPE, STRIPE)],
                    num_out.at[c, pl.ds(s * STRIPE, STRIPE)])


_sc_call = functools.partial(
    pl.kernel,
    out_type=(
        jax.ShapeDtypeStruct((NC, NPAD, H), jnp.float32),
        jax.ShapeDtypeStruct((DEN_B, NW, 1024), jnp.float32),
    ),
    mesh=plsc.VectorSubcoreMesh(core_axis_name="c", subcore_axis_name="s",
                                num_cores=NC, num_subcores=NS),
    compiler_params=pltpu.CompilerParams(needs_layout_passes=False),
    scratch_types=[
        pltpu.VMEM((N,), jnp.float32),
        pltpu.VMEM((N,), jnp.float32),
        pltpu.VMEM((IDXW,), jnp.int32),
        pltpu.VMEM((IDXW,), jnp.int32),
        pltpu.VMEM((IDXW,), jnp.int32),
        pltpu.VMEM((IDXW,), jnp.int32),
        pltpu.VMEM((CH,), jnp.int32),
        pltpu.VMEM((CH,), jnp.int32),
        pltpu.VMEM((CH,), jnp.int32),
        pltpu.VMEM((CH,), jnp.int32),
        pltpu.VMEM((CH,), jnp.float32),
        pltpu.VMEM((DEN_B, 1024), jnp.float32),
        pltpu.VMEM((CH, H), jnp.float32),
        pltpu.VMEM((CH, H), jnp.float32),
        pltpu.VMEM((2 * H,), jnp.float32),
        pltpu.VMEM_SHARED((NPAD, H), jnp.float32),
        pltpu.SemaphoreType.DMA,
        pltpu.SemaphoreType.DMA,
        pltpu.SemaphoreType.DMA,
        pltpu.SemaphoreType.DMA,
        pltpu.SemaphoreType.DMA,
    ],
)(_sc_body)


def _fin_body(num_ref, den_ref, bias_ref, out_ref):
    n = num_ref[0] + num_ref[1]
    dsum = jnp.sum(den_ref[0], axis=0)
    out_ref[...] = jnp.maximum(
        n / (dsum[:, None] + 1e-16) + bias_ref[...], 0.0)


_fin_call = pl.pallas_call(
    _fin_body,
    grid=(NFB,),
    in_specs=[
        pl.BlockSpec((NC, FRB, H), lambda i: (0, i, 0)),
        pl.BlockSpec((1, NW, FRB), lambda i: (i // 8, 0, i % 8)),
        pl.BlockSpec((1, H), lambda i: (0, 0)),
    ],
    out_specs=pl.BlockSpec((FRB, H), lambda i: (i, 0)),
    out_shape=jax.ShapeDtypeStruct((N, H), jnp.float32),
)


def kernel(x, edge_index, W_src, W_dst, att_src, att_dst, bias):
    xs, a_src, a_dst, mb2 = _proj_call(
        x, W_src, W_dst, att_src.reshape(1, H), att_dst.reshape(1, H))
    src = edge_index[0].reshape(NW, NSB, IDXW)
    dst = edge_index[1].reshape(NW, NSB, IDXW)
    num, den = _sc_call(a_src.reshape(N), a_dst.reshape(N), src, dst, xs,
                        mb2.reshape(2 * H))
    return _fin_call(num, den, bias.reshape(1, H))
